# baseline (speedup 1.0000x reference)
import functools

import jax
import jax.numpy as jnp
from jax import lax
from jax.experimental import pallas as pl
from jax.experimental.pallas import tpu as pltpu
from jax.experimental.pallas import tpu_sc as plsc

B, F = 128, 32768
K = F // 2
NC, NS, L = 2, 16, 16
NW = NC * NS
RPW = B // NW
H1BITS = 11
H1SHIFT = 31 - H1BITS
NB1 = 1 << H1BITS
VPASS = F // L


def _row_threshold(buf, cand_ref, hist_ref):
    iot = lax.iota(jnp.int32, L)
    UN = 24

    @plsc.parallel_loop(0, NB1 // L, 1, unroll=8)
    def zb(g):
        hist_ref[pl.ds(g * L, L)] = jnp.zeros((L,), jnp.int32)

    ones = jnp.ones((L,), jnp.int32)

    @plsc.parallel_loop(0, VPASS, 1, unroll=UN)
    def p1(i):
        v = buf[pl.ds(i * L, L)]
        b1 = (lax.bitcast_convert_type(v, jnp.int32)
              & jnp.int32(0x7FFFFFFF)) >> H1SHIFT
        plsc.addupdate_scatter(hist_ref, [b1], ones)

    def wcond(s):
        _, run, nxt = s
        return run + nxt <= K

    def wbody(s):
        g, run, nxt = s
        g = g + 1
        return g, run + nxt, jnp.sum(hist_ref[pl.ds(g * L, L)])

    g0 = jnp.int32(0)
    s = (g0, jnp.int32(0), jnp.sum(hist_ref[pl.ds(0, L)]))
    gs, base, _ = lax.while_loop(wcond, wbody, s)

    v = hist_ref[pl.ds(gs * L, L)]
    c = plsc.cumsum(v)
    cexcl = c - v
    first = ((base + c > K) & (base + cexcl <= K)).astype(jnp.int32)
    b1s = jnp.sum(first * (gs * L + iot))
    kp = jnp.sum(first * (K - base - cexcl))

    @plsc.parallel_loop(0, VPASS, 1, unroll=UN, carry=jnp.int32(0))
    def p2(i, off):
        w = buf[pl.ds(i * L, L)]
        bits = lax.bitcast_convert_type(w, jnp.int32) & jnp.int32(0x7FFFFFFF)
        sel = (bits >> H1SHIFT) == b1s
        plsc.store_compressed(cand_ref.at[pl.ds(off, L)], bits, mask=sel)
        return off + plsc.all_reduce_population_count(sel)[0]
    m = p2

    for u in range(2 * UN):
        cand_ref[pl.ds(m + u * L, L)] = jnp.full((L,), 0x7FFFFFFF, jnp.int32)
    nv8 = (m + UN * L - 1) // (UN * L)

    def bs(i, r):
        cbit = r | (jnp.int32(1) << (H1SHIFT - 1 - i))

        def cl(jj, accs):
            ws = [cand_ref[pl.ds((jj * UN + u) * L, L)] for u in range(UN)]
            lts = [(w < cbit).astype(jnp.int32) for w in ws]
            return tuple(a + x for a, x in zip(accs, lts))
        accs = lax.fori_loop(
            0, nv8, cl, tuple(jnp.zeros((L,), jnp.int32) for _ in range(UN)))
        cnt = jnp.sum(sum(accs))
        return jnp.where(cnt <= kp, cbit, r)
    return lax.fori_loop(0, H1SHIFT, bs, b1s << H1SHIFT)


def _sc_thresholds(x):
    mesh = plsc.VectorSubcoreMesh(core_axis_name="c", subcore_axis_name="s")

    @functools.partial(
        pl.kernel,
        out_type=jax.ShapeDtypeStruct((NW, L), jnp.float32),
        mesh=mesh,
        compiler_params=pltpu.CompilerParams(
            needs_layout_passes=False, use_tc_tiling_on_sc=True),
        scratch_types=[
            pltpu.VMEM((F,), jnp.float32),
            pltpu.VMEM((F,), jnp.float32),
            pltpu.VMEM((F + 16 * L,), jnp.int32),
            pltpu.VMEM((NB1,), jnp.int32),
            pltpu.VMEM((L,), jnp.float32),
            pltpu.SemaphoreType.DMA,
            pltpu.SemaphoreType.DMA,
        ],
    )
    def k(x_hbm, t_hbm, rowa, rowb, cand, hist, tv, sema, semb):
        wid = lax.axis_index("s") * NC + lax.axis_index("c")
        row0 = wid * RPW
        bufs, sems = (rowa, rowb), (sema, semb)
        pltpu.async_copy(x_hbm.at[row0], rowa, sema)
        tvec = jnp.zeros((L,), jnp.int32)
        for j in range(RPW):
            pltpu.make_async_copy(x_hbm.at[row0 + j], bufs[j % 2], sems[j % 2]).wait()
            if j + 1 < RPW:
                pltpu.async_copy(
                    x_hbm.at[row0 + j + 1], bufs[(j + 1) % 2], sems[(j + 1) % 2])
            tbits = _row_threshold(bufs[j % 2], cand, hist)
            tvec = jnp.where(lax.iota(jnp.int32, L) == j, tbits, tvec)
        tv[...] = lax.bitcast_convert_type(tvec, jnp.float32)
        pltpu.sync_copy(tv, t_hbm.at[wid])

    out = k(x)
    return out[:, :RPW].reshape(B, 1)


BB = 32


def _eval_body(x_ref, t_ref, o_ref):
    a = jnp.abs(x_ref[...])
    t = t_ref[...]
    m = jnp.where(a > t, a, 0.0)
    for u in range(BB):
        o_ref[u] = m[:, u * 256:(u + 1) * 256].T.reshape(F)


def _eval_out(x, t):
    return pl.pallas_call(
        _eval_body,
        grid=(B // BB,),
        in_specs=[
            pl.BlockSpec((B, BB * 256), lambda b: (0, b)),
            pl.BlockSpec((B, 1), lambda b: (0, 0)),
        ],
        out_specs=pl.BlockSpec((BB, F), lambda b: (b, 0)),
        out_shape=jax.ShapeDtypeStruct((B, F), jnp.float32),
    )(x, t)


def _train_body(x_ref, t_ref, m2_ref, o_ref):
    a = jnp.abs(x_ref[...])
    t = t_ref[...]
    drop = (a <= t) & (m2_ref[...] != 0)
    o_ref[...] = jnp.where(drop, 0.0, a)


def _train_out(x, t):
    u = jax.random.uniform(
        jax.random.fold_in(jax.random.key(0), 1), (F, B), dtype=jnp.float32)
    m2 = (u <= 0.5).T.astype(jnp.float32)
    return pl.pallas_call(
        _train_body,
        grid=(8,),
        in_specs=[
            pl.BlockSpec((B // 8, F), lambda i: (i, 0)),
            pl.BlockSpec((B // 8, 1), lambda i: (i, 0)),
            pl.BlockSpec((B // 8, F), lambda i: (i, 0)),
        ],
        out_specs=pl.BlockSpec((B // 8, F), lambda i: (i, 0)),
        out_shape=jax.ShapeDtypeStruct((B, F), jnp.float32),
    )(x, t, m2)


def kernel(input, is_training):
    x = input.reshape(B, F)
    t = _sc_thresholds(x)
    out = lax.cond(
        jnp.asarray(is_training) == 0,
        lambda x, t: _eval_out(x, t),
        lambda x, t: _train_out(x, t),
        x, t)
    return out.reshape(input.shape)


# speedup vs baseline: 1.0708x; 1.0708x over previous
import functools

import jax
import jax.numpy as jnp
from jax import lax
from jax.experimental import pallas as pl
from jax.experimental.pallas import tpu as pltpu
from jax.experimental.pallas import tpu_sc as plsc

B, F = 128, 32768
K = F // 2
NC, NS, L = 2, 16, 16
NW = NC * NS
RPW = B // NW
H1BITS = 11
H1SHIFT = 31 - H1BITS
NB1 = 1 << H1BITS
VPASS = F // L


def _row_threshold(buf, cand_ref, hist_ref):
    iot = lax.iota(jnp.int32, L)
    UN = 16

    @plsc.parallel_loop(0, NB1 // L, 1, unroll=8)
    def zb(g):
        hist_ref[pl.ds(g * L, L)] = jnp.zeros((L,), jnp.int32)

    ones = jnp.ones((L,), jnp.int32)

    @plsc.parallel_loop(0, VPASS, 1, unroll=UN)
    def p1(i):
        v = buf[pl.ds(i * L, L)]
        b1 = (lax.bitcast_convert_type(v, jnp.int32)
              & jnp.int32(0x7FFFFFFF)) >> H1SHIFT
        plsc.addupdate_scatter(hist_ref, [b1], ones)

    def wcond(s):
        _, run, nxt = s
        return run + nxt <= K

    def wbody(s):
        g, run, nxt = s
        g = g + 1
        return g, run + nxt, jnp.sum(hist_ref[pl.ds(g * L, L)])

    g0 = jnp.int32(0)
    s = (g0, jnp.int32(0), jnp.sum(hist_ref[pl.ds(0, L)]))
    gs, base, _ = lax.while_loop(wcond, wbody, s)

    v = hist_ref[pl.ds(gs * L, L)]
    c = plsc.cumsum(v)
    cexcl = c - v
    first = ((base + c > K) & (base + cexcl <= K)).astype(jnp.int32)
    b1s = jnp.sum(first * (gs * L + iot))
    kp = jnp.sum(first * (K - base - cexcl))

    @plsc.parallel_loop(0, VPASS, 1, unroll=UN, carry=jnp.int32(0))
    def p2(i, off):
        w = buf[pl.ds(i * L, L)]
        bits = lax.bitcast_convert_type(w, jnp.int32) & jnp.int32(0x7FFFFFFF)
        sel = (bits >> H1SHIFT) == b1s
        plsc.store_compressed(cand_ref.at[pl.ds(off, L)], bits, mask=sel)
        return off + plsc.all_reduce_population_count(sel)[0]
    m = p2

    for u in range(2 * UN):
        cand_ref[pl.ds(m + u * L, L)] = jnp.full((L,), 0x7FFFFFFF, jnp.int32)
    nv8 = (m + UN * L - 1) // (UN * L)

    def bs(i, r):
        cbit = r | (jnp.int32(1) << (H1SHIFT - 1 - i))

        def cl(jj, accs):
            ws = [cand_ref[pl.ds((jj * UN + u) * L, L)] for u in range(UN)]
            lts = [(w < cbit).astype(jnp.int32) for w in ws]
            return tuple(a + x for a, x in zip(accs, lts))
        accs = lax.fori_loop(
            0, nv8, cl, tuple(jnp.zeros((L,), jnp.int32) for _ in range(UN)))
        cnt = jnp.sum(sum(accs))
        return jnp.where(cnt <= kp, cbit, r)
    return lax.fori_loop(0, H1SHIFT, bs, b1s << H1SHIFT)


def _sc_thresholds(x):
    mesh = plsc.VectorSubcoreMesh(core_axis_name="c", subcore_axis_name="s")

    @functools.partial(
        pl.kernel,
        out_type=jax.ShapeDtypeStruct((NW, L), jnp.float32),
        mesh=mesh,
        compiler_params=pltpu.CompilerParams(
            needs_layout_passes=False, use_tc_tiling_on_sc=True),
        scratch_types=[
            pltpu.VMEM((F,), jnp.float32),
            pltpu.VMEM((F,), jnp.float32),
            pltpu.VMEM((F + 16 * L,), jnp.int32),
            pltpu.VMEM((NB1,), jnp.int32),
            pltpu.VMEM((L,), jnp.float32),
            pltpu.SemaphoreType.DMA,
            pltpu.SemaphoreType.DMA,
        ],
    )
    def k(x_hbm, t_hbm, rowa, rowb, cand, hist, tv, sema, semb):
        wid = lax.axis_index("s") * NC + lax.axis_index("c")
        row0 = wid * RPW
        bufs, sems = (rowa, rowb), (sema, semb)
        pltpu.async_copy(x_hbm.at[row0], rowa, sema)
        tvec = jnp.zeros((L,), jnp.int32)
        for j in range(RPW):
            pltpu.make_async_copy(x_hbm.at[row0 + j], bufs[j % 2], sems[j % 2]).wait()
            if j + 1 < RPW:
                pltpu.async_copy(
                    x_hbm.at[row0 + j + 1], bufs[(j + 1) % 2], sems[(j + 1) % 2])
            tbits = _row_threshold(bufs[j % 2], cand, hist)
            tvec = jnp.where(lax.iota(jnp.int32, L) == j, tbits, tvec)
        tv[...] = lax.bitcast_convert_type(tvec, jnp.float32)
        pltpu.sync_copy(tv, t_hbm.at[wid])

    out = k(x)
    return out[:, :RPW].reshape(B, 1)


BB = 32


def _eval_body(x_ref, t_ref, o_ref):
    a = jnp.abs(x_ref[...])
    t = t_ref[...]
    m = jnp.where(a > t, a, 0.0)
    for u in range(BB):
        o_ref[u] = m[:, u * 256:(u + 1) * 256].T.reshape(F)


def _eval_out(x, t):
    return pl.pallas_call(
        _eval_body,
        grid=(B // BB,),
        in_specs=[
            pl.BlockSpec((B, BB * 256), lambda b: (0, b)),
            pl.BlockSpec((B, 1), lambda b: (0, 0)),
        ],
        out_specs=pl.BlockSpec((BB, F), lambda b: (b, 0)),
        out_shape=jax.ShapeDtypeStruct((B, F), jnp.float32),
    )(x, t)


def _train_body(x_ref, t_ref, m2_ref, o_ref):
    a = jnp.abs(x_ref[...])
    t = t_ref[...]
    drop = (a <= t) & (m2_ref[...] != 0)
    o_ref[...] = jnp.where(drop, 0.0, a)


def _train_out(x, t):
    u = jax.random.uniform(
        jax.random.fold_in(jax.random.key(0), 1), (F, B), dtype=jnp.float32)
    m2 = (u <= 0.5).T.astype(jnp.float32)
    return pl.pallas_call(
        _train_body,
        grid=(8,),
        in_specs=[
            pl.BlockSpec((B // 8, F), lambda i: (i, 0)),
            pl.BlockSpec((B // 8, 1), lambda i: (i, 0)),
            pl.BlockSpec((B // 8, F), lambda i: (i, 0)),
        ],
        out_specs=pl.BlockSpec((B // 8, F), lambda i: (i, 0)),
        out_shape=jax.ShapeDtypeStruct((B, F), jnp.float32),
    )(x, t, m2)


def kernel(input, is_training):
    x = input.reshape(B, F)
    t = _sc_thresholds(x)
    out = lax.cond(
        jnp.asarray(is_training) == 0,
        lambda x, t: _eval_out(x, t),
        lambda x, t: _train_out(x, t),
        x, t)
    return out.reshape(input.shape)
